# baseline (speedup 1.0000x reference)
import functools

import jax
import jax.numpy as jnp
from jax import lax
from jax.experimental import pallas as pl
from jax.experimental.pallas import tpu as pltpu
from jax.experimental.pallas import tpu_sc as plsc

F32 = jnp.float32
I32 = jnp.int32

_NUM_CORES = 2
_NUM_TILES = 16


def _mm1t_body(w_ref, q_ref, o_ref):
    o_ref[...] = jnp.dot(w_ref[...], q_ref[0],
                         preferred_element_type=F32)[None]


def _mm1t(qT, W1T):
    NB, D, N = qT.shape
    return pl.pallas_call(
        _mm1t_body,
        grid=(NB,),
        in_specs=[pl.BlockSpec((D, D), lambda b: (0, 0)),
                  pl.BlockSpec((1, D, N), lambda b: (b, 0, 0))],
        out_specs=pl.BlockSpec((1, D, N), lambda b: (b, 0, 0)),
        out_shape=jax.ShapeDtypeStruct((NB, D, N), F32),
    )(W1T, qT)


def _sc1_build(NB, NP, DM, N, E_pad):
    FP = 4
    NPASS = DM // (_NUM_TILES * FP)
    K = 2048
    NBLK = E_pad // K
    NCH = K // 16
    STRIPE = NP // _NUM_TILES
    mesh = plsc.VectorSubcoreMesh(core_axis_name="c", subcore_axis_name="s",
                                  num_cores=_NUM_CORES, num_subcores=_NUM_TILES)

    @functools.partial(
        pl.kernel,
        out_type=jax.ShapeDtypeStruct((NB * NP,), F32),
        mesh=mesh,
        compiler_params=pltpu.CompilerParams(needs_layout_passes=False),
        scratch_types=[
            pltpu.VMEM((FP * N,), F32),
            pltpu.VMEM((FP * NP,), F32),
            pltpu.VMEM((K,), I32),
            pltpu.VMEM((K,), I32),
            pltpu.VMEM((NP,), F32),
            pltpu.VMEM((_NUM_TILES * STRIPE,), F32),
            pltpu.VMEM((DM + 16,), F32),
            pltpu.VMEM((DM + 16,), F32),
            pltpu.VMEM_SHARED((_NUM_TILES * NP,), F32),
        ],
    )
    def k(ht_hbm, src_hbm, dst_hbm, z_hbm, b1_hbm, w2_hbm, out_hbm,
          htab, acc, srcblk, dstblk, spart, redbuf, b1v, w2v, shared):
        c = lax.axis_index("c")
        s = lax.axis_index("s")
        pltpu.sync_copy(b1_hbm, b1v)
        pltpu.sync_copy(w2_hbm, w2v)
        i16 = lax.iota(I32, 16)
        for bb in range(NB // _NUM_CORES):
            b = c * (NB // _NUM_CORES) + bb
            pltpu.sync_copy(z_hbm.at[pl.ds(0, NP)], spart)
            for p in range(NPASS):
                f0 = p * (_NUM_TILES * FP) + s * FP
                pltpu.sync_copy(
                    ht_hbm.at[pl.ds((b * DM + f0) * N, FP * N)], htab)
                pltpu.sync_copy(z_hbm.at[pl.ds(0, FP * NP)], acc)

                def eblock(eb, carry):
                    base = pl.multiple_of(eb * K, 8)
                    pltpu.sync_copy(src_hbm.at[pl.ds(base, K)], srcblk)
                    pltpu.sync_copy(dst_hbm.at[pl.ds(base, K)], dstblk)

                    def chunk16(i, carry2):
                        s16 = srcblk[pl.ds(i * 16, 16)]
                        d16 = dstblk[pl.ds(i * 16, 16)]
                        for fl in range(FP):
                            v = plsc.load_gather(htab, [s16 + fl * N])
                            plsc.addupdate_scatter(acc, [d16 + fl * NP], v)
                        return carry2

                    lax.fori_loop(0, NCH, chunk16, 0)
                    return carry

                lax.fori_loop(0, NBLK, eblock, 0)

                b1g = plsc.load_gather(b1v, [f0 + i16])
                w2g = plsc.load_gather(w2v, [f0 + i16])

                def snode(nb, carry):
                    tot = spart[pl.ds(nb * 16, 16)]
                    for fl in range(FP):
                        x = acc[pl.ds(fl * NP + nb * 16, 16)] + b1g[fl]
                        x = jnp.clip(x, -20.0, 20.0)
                        e = jnp.exp(x + x)
                        tot = tot + ((e - 1.0) / (e + 1.0)) * w2g[fl]
                    spart[pl.ds(nb * 16, 16)] = tot
                    return carry

                lax.fori_loop(0, NP // 16, snode, 0)

            pltpu.sync_copy(spart, shared.at[pl.ds(s * NP, NP)])
            plsc.subcore_barrier()
            for t in range(_NUM_TILES):
                pltpu.sync_copy(shared.at[pl.ds(t * NP + s * STRIPE, STRIPE)],
                                redbuf.at[pl.ds(t * STRIPE, STRIPE)])

            def sred(j, carry):
                tot = redbuf[pl.ds(j * 16, 16)]
                for t in range(1, _NUM_TILES):
                    tot = tot + redbuf[pl.ds(t * STRIPE + j * 16, 16)]
                spart[pl.ds(j * 16, 16)] = tot
                return carry

            lax.fori_loop(0, STRIPE // 16, sred, 0)
            pltpu.sync_copy(spart.at[pl.ds(0, STRIPE)],
                            out_hbm.at[pl.ds(b * NP + s * STRIPE, STRIPE)])
            plsc.subcore_barrier()

    return k


def _sc2_build(NB, NP, EPT):
    NPB = NP * NB
    NCH = EPT // 16
    NW = _NUM_CORES * _NUM_TILES
    mesh = plsc.VectorSubcoreMesh(core_axis_name="c", subcore_axis_name="s",
                                  num_cores=_NUM_CORES, num_subcores=_NUM_TILES)

    @functools.partial(
        pl.kernel,
        out_type=jax.ShapeDtypeStruct((NW * NPB,), F32),
        mesh=mesh,
        compiler_params=pltpu.CompilerParams(needs_layout_passes=False),
        scratch_types=[
            pltpu.VMEM((NPB,), F32),
            pltpu.VMEM((NPB,), F32),
            pltpu.VMEM((EPT,), I32),
            pltpu.VMEM((EPT,), I32),
        ],
    )
    def k(s_hbm, src_hbm, dst_hbm, z_hbm, out_hbm,
          s_loc, acc, src_loc, dst_loc):
        c = lax.axis_index("c")
        s = lax.axis_index("s")
        w = c * _NUM_TILES + s
        pltpu.sync_copy(s_hbm, s_loc)
        pltpu.sync_copy(z_hbm, acc)
        pltpu.sync_copy(src_hbm.at[pl.ds(w * EPT, EPT)], src_loc)
        pltpu.sync_copy(dst_hbm.at[pl.ds(w * EPT, EPT)], dst_loc)

        def body(i, carry):
            sv = src_loc[pl.ds(i * 16, 16)] * NB
            dv = dst_loc[pl.ds(i * 16, 16)] * NB
            for b in range(NB):
                vals = plsc.load_gather(s_loc, [sv + b])
                plsc.addupdate_scatter(acc, [dv + b], vals)
            return carry

        lax.fori_loop(0, NCH, body, 0)
        pltpu.sync_copy(acc, out_hbm.at[pl.ds(w * NPB, NPB)])

    return k


def _att_body(NB, NVALID, pt_ref, b2_ref, v_ref, ctx_ref, sc_ref):
    NW = pt_ref.shape[0]
    sp = pt_ref[0] + b2_ref[...]
    for t in range(1, NW):
        sp = sp + pt_ref[t]
    sc_ref[...] = sp
    col = lax.broadcasted_iota(I32, sp.shape, 1)
    valid = col < NVALID
    m = jnp.max(jnp.where(valid, sp, -1e30), axis=1, keepdims=True)
    e = jnp.where(valid, jnp.exp(sp - m), 0.0)
    z = jnp.sum(e, axis=1, keepdims=True)
    attn = e / z
    ctx_ref[...] = jnp.concatenate(
        [jnp.dot(attn[b:b + 1, :NVALID], v_ref[b],
                 preferred_element_type=F32,
                 precision=lax.Precision.HIGHEST)
         for b in range(NB)], axis=0)


def _att(Pt, b2, values):
    NB, N, D = values.shape
    NW, NP = Pt.shape[0], Pt.shape[2]
    return pl.pallas_call(
        functools.partial(_att_body, NB, N),
        grid=(1,),
        in_specs=[pl.BlockSpec((NW, NB, NP), lambda i: (0, 0, 0)),
                  pl.BlockSpec((1, 1), lambda i: (0, 0)),
                  pl.BlockSpec((NB, N, D), lambda i: (0, 0, 0))],
        out_specs=[pl.BlockSpec((NB, D), lambda i: (0, 0)),
                   pl.BlockSpec((NB, NP), lambda i: (0, 0))],
        out_shape=[jax.ShapeDtypeStruct((NB, D), F32),
                   jax.ShapeDtypeStruct((NB, NP), F32)],
    )(Pt, b2.reshape(1, 1), values)


def kernel(query, values, edges, W1, b1, W2, b2):
    NB, N, D = query.shape
    E = edges.shape[1]

    NP = -(-(N + 1) // 2048) * 2048
    STRIPE = NP // _NUM_TILES
    G1, G2 = 128, 64
    EPT1 = -(-E // (_NUM_TILES * G1)) * G1
    if (EPT1 // G1) % 2:
        EPT1 += G1
    E_pad = EPT1 * _NUM_TILES
    NG1 = EPT1 // G1
    EPT2 = E_pad // (_NUM_TILES * _NUM_CORES)
    NG2 = EPT2 // G2

    src = edges[0]
    dst = edges[1]
    pad = E_pad - E
    src_p = jnp.concatenate([src, jnp.arange(pad, dtype=I32) % N])
    dst_p = jnp.concatenate([dst, N + (jnp.arange(pad, dtype=I32) % (NP - N))])
    z4 = jnp.zeros((NP * NB,), F32)
    NW = _NUM_CORES * _NUM_TILES

    qT = jnp.transpose(query, (0, 2, 1))
    HT = _mm1t(qT, W1.T)
    b1p = jnp.concatenate([b1, jnp.zeros((16,), F32)])
    w2p = jnp.concatenate([W2.reshape(D), jnp.zeros((16,), F32)])
    Sflat = _sc1_build(NB, NP, D, N, E_pad)(
        HT.reshape(NB * D * N), src_p, dst_p, z4, b1p, w2p)
    S4 = Sflat.reshape(NB, NP).T.reshape(NP * NB)
    P = _sc2_build(NB, NP, EPT2)(S4, src_p, dst_p, z4)
    Pt = jnp.transpose(P.reshape(NW, NP, NB), (0, 2, 1))
    ctx, scores = _att(Pt, b2, values)
    return ctx, scores[:, :N, None]


# speedup vs baseline: 2.8240x; 2.8240x over previous
import functools

import jax
import jax.numpy as jnp
from jax import lax
from jax.experimental import pallas as pl
from jax.experimental.pallas import tpu as pltpu
from jax.experimental.pallas import tpu_sc as plsc

F32 = jnp.float32
I32 = jnp.int32

_NUM_CORES = 2
_NUM_TILES = 16


def _mm1_body(q_ref, w_ref, o_ref):
    o_ref[...] = jnp.dot(q_ref[...], w_ref[...], preferred_element_type=F32)


def _mm1(qflat, W1):
    M, D = qflat.shape
    BLK = 2000
    return pl.pallas_call(
        _mm1_body,
        grid=(M // BLK,),
        in_specs=[pl.BlockSpec((BLK, D), lambda i: (i, 0)),
                  pl.BlockSpec((D, W1.shape[1]), lambda i: (0, 0))],
        out_specs=pl.BlockSpec((BLK, W1.shape[1]), lambda i: (i, 0)),
        out_shape=jax.ShapeDtypeStruct((M, W1.shape[1]), F32),
    )(qflat, W1)


def _sc1_build(NB, NP, DM, EPT, NG, G, STRIPE):
    mesh = plsc.VectorSubcoreMesh(core_axis_name="c", subcore_axis_name="s",
                                  num_cores=_NUM_CORES, num_subcores=_NUM_TILES)

    @functools.partial(
        pl.kernel,
        out_type=jax.ShapeDtypeStruct((NB, NP, DM), F32),
        mesh=mesh,
        scratch_types=[
            pltpu.VMEM((G,), I32),
            pltpu.VMEM((G,), I32),
            pltpu.VMEM((G,), I32),
            pltpu.VMEM((G,), I32),
            pltpu.VMEM((G, DM), F32),
            pltpu.VMEM((G, DM), F32),
            pltpu.VMEM_SHARED((NP, DM), F32),
            pltpu.SemaphoreType.DMA,
            pltpu.SemaphoreType.DMA,
            pltpu.SemaphoreType.DMA,
            pltpu.SemaphoreType.DMA,
        ],
    )
    def k(h_hbm, srcb_hbm, dst_hbm, z_hbm, out_hbm,
          isA, idA, isB, idB, rowsA, rowsB, acc, gA, gB, sA, sB):
        c = lax.axis_index("c")
        s = lax.axis_index("s")
        row0 = s * STRIPE
        for bb in range(NB // _NUM_CORES):
            b = c * (NB // _NUM_CORES) + bb
            pltpu.sync_copy(z_hbm, acc.at[pl.ds(row0, STRIPE)])
            plsc.subcore_barrier()

            def body(j, carry):
                base = pl.multiple_of(s * EPT + j * (2 * G), 8)

                @pl.when(j > 0)
                def _drainA():
                    pltpu.make_async_copy(z_hbm.at[pl.ds(0, G)], rowsA, sA).wait()

                pltpu.sync_copy(srcb_hbm.at[b, pl.ds(base, G)], isA)
                pltpu.sync_copy(dst_hbm.at[pl.ds(base, G)], idA)
                pltpu.async_copy(h_hbm.at[isA], rowsA, gA).wait()
                pltpu.async_copy(rowsA, acc.at[idA], sA, add=True)

                @pl.when(j > 0)
                def _drainB():
                    pltpu.make_async_copy(z_hbm.at[pl.ds(0, G)], rowsB, sB).wait()

                pltpu.sync_copy(srcb_hbm.at[b, pl.ds(base + G, G)], isB)
                pltpu.sync_copy(dst_hbm.at[pl.ds(base + G, G)], idB)
                pltpu.async_copy(h_hbm.at[isB], rowsB, gB).wait()
                pltpu.async_copy(rowsB, acc.at[idB], sB, add=True)
                return carry

            lax.fori_loop(0, NG // 2, body, 0)
            pltpu.make_async_copy(z_hbm.at[pl.ds(0, G)], rowsA, sA).wait()
            pltpu.make_async_copy(z_hbm.at[pl.ds(0, G)], rowsB, sB).wait()
            plsc.subcore_barrier()
            pltpu.sync_copy(acc.at[pl.ds(row0, STRIPE)],
                            out_hbm.at[b, pl.ds(row0, STRIPE)])

    return k


def _mm2_body(a_ref, b1_ref, w_ref, o_ref):
    t = jnp.tanh(a_ref[...] + b1_ref[...])
    o_ref[...] = jnp.sum(t * w_ref[...], axis=1, keepdims=True)


def _mm2(aggflat, b1, W2):
    M, D = aggflat.shape
    BLK = next(b for b in (4096, 2048, 1024, 512, 256, 128, 64, 32, 16, 8)
               if M % b == 0)
    return pl.pallas_call(
        _mm2_body,
        grid=(M // BLK,),
        in_specs=[pl.BlockSpec((BLK, D), lambda i: (i, 0)),
                  pl.BlockSpec((1, D), lambda i: (0, 0)),
                  pl.BlockSpec((1, D), lambda i: (0, 0))],
        out_specs=pl.BlockSpec((BLK, 1), lambda i: (i, 0)),
        out_shape=jax.ShapeDtypeStruct((M, 1), F32),
    )(aggflat, b1.reshape(1, D), W2.reshape(1, D))


def _sc2_build(NB, NP, EPT):
    NPB = NP * NB
    NCH = EPT // 16
    NW = _NUM_CORES * _NUM_TILES
    mesh = plsc.VectorSubcoreMesh(core_axis_name="c", subcore_axis_name="s",
                                  num_cores=_NUM_CORES, num_subcores=_NUM_TILES)

    @functools.partial(
        pl.kernel,
        out_type=jax.ShapeDtypeStruct((NW * NPB,), F32),
        mesh=mesh,
        compiler_params=pltpu.CompilerParams(needs_layout_passes=False),
        scratch_types=[
            pltpu.VMEM((NPB,), F32),
            pltpu.VMEM((NPB,), F32),
            pltpu.VMEM((EPT,), I32),
            pltpu.VMEM((EPT,), I32),
        ],
    )
    def k(s_hbm, src_hbm, dst_hbm, z_hbm, out_hbm,
          s_loc, acc, src_loc, dst_loc):
        c = lax.axis_index("c")
        s = lax.axis_index("s")
        w = c * _NUM_TILES + s
        pltpu.sync_copy(s_hbm, s_loc)
        pltpu.sync_copy(z_hbm, acc)
        pltpu.sync_copy(src_hbm.at[pl.ds(w * EPT, EPT)], src_loc)
        pltpu.sync_copy(dst_hbm.at[pl.ds(w * EPT, EPT)], dst_loc)

        def body(i, carry):
            sv = src_loc[pl.ds(i * 16, 16)] * NB
            dv = dst_loc[pl.ds(i * 16, 16)] * NB
            for b in range(NB):
                vals = plsc.load_gather(s_loc, [sv + b])
                plsc.addupdate_scatter(acc, [dv + b], vals)
            return carry

        lax.fori_loop(0, NCH, body, 0)
        pltpu.sync_copy(acc, out_hbm.at[pl.ds(w * NPB, NPB)])

    return k


def _att_body(NVALID, pt_ref, b2_ref, v_ref, ctx_ref, sc_ref):
    NW = pt_ref.shape[0]
    sp = pt_ref[0] + b2_ref[...]
    for t in range(1, NW):
        sp = sp + pt_ref[t]
    sp = jnp.broadcast_to(sp[:1], sc_ref.shape)
    sc_ref[...] = sp
    col = lax.broadcasted_iota(I32, sp.shape, 1)
    valid = col < NVALID
    m = jnp.max(jnp.where(valid, sp, -1e30), axis=1, keepdims=True)
    e = jnp.where(valid, jnp.exp(sp - m), 0.0)
    z = jnp.sum(e, axis=1, keepdims=True)
    attn = e / z
    ctx_ref[...] = jnp.dot(attn[:, :NVALID], v_ref[0],
                           preferred_element_type=F32,
                           precision=lax.Precision.HIGHEST)


def _att(Pt2, b2, values):
    NB, N, D = values.shape
    NW = Pt2.shape[0]
    NP = Pt2.shape[1] // NB
    ctx8, sc8 = pl.pallas_call(
        functools.partial(_att_body, N),
        grid=(NB,),
        in_specs=[pl.BlockSpec((NW, NP), lambda b: (0, b)),
                  pl.BlockSpec((1, 1), lambda b: (0, 0)),
                  pl.BlockSpec((1, N, D), lambda b: (b, 0, 0))],
        out_specs=[pl.BlockSpec((8, D), lambda b: (b, 0)),
                   pl.BlockSpec((8, NP), lambda b: (b, 0))],
        out_shape=[jax.ShapeDtypeStruct((NB * 8, D), F32),
                   jax.ShapeDtypeStruct((NB * 8, NP), F32)],
    )(Pt2, b2.reshape(1, 1), values)
    return ctx8.reshape(NB, 8, D)[:, 0], sc8.reshape(NB, 8, NP)[:, 0]


def kernel(query, values, edges, W1, b1, W2, b2):
    NB, N, D = query.shape
    E = edges.shape[1]

    NP = -(-(N + 1) // 2048) * 2048
    STRIPE = NP // _NUM_TILES
    G1, G2 = 128, 64
    EPT1 = -(-E // (_NUM_TILES * G1)) * G1
    if (EPT1 // G1) % 2:
        EPT1 += G1
    E_pad = EPT1 * _NUM_TILES
    NG1 = EPT1 // G1
    EPT2 = E_pad // (_NUM_TILES * _NUM_CORES)
    NG2 = EPT2 // G2

    src = edges[0]
    dst = edges[1]
    pad = E_pad - E
    src_p = jnp.concatenate([src, jnp.arange(pad, dtype=I32) % N])
    dst_p = jnp.concatenate([dst, N + (jnp.arange(pad, dtype=I32) % (NP - N))])
    srcb = src_p[None, :] + (jnp.arange(NB, dtype=I32) * N)[:, None]
    z1 = jnp.zeros((STRIPE, D), F32)
    z4 = jnp.zeros((NP * NB,), F32)
    NW = _NUM_CORES * _NUM_TILES

    Hflat = _mm1(query.reshape(NB * N, D), W1)
    AGG = _sc1_build(NB, NP, D, EPT1, NG1, G1, STRIPE)(
        Hflat, srcb, dst_p, z1)
    Sv = _mm2(AGG.reshape(NB * NP, D), b1, W2)
    S4 = Sv.reshape(NB, NP).T.reshape(NP * NB)
    P = _sc2_build(NB, NP, EPT2)(S4, src_p, dst_p, z4)
    Pt2 = jnp.transpose(P.reshape(NW, NP, NB),
                        (0, 2, 1)).reshape(NW, NB * NP)
    ctx, scores = _att(Pt2, b2, values)
    return ctx, scores[:, :N, None]


# speedup vs baseline: 3.1426x; 1.1128x over previous
import functools

import jax
import jax.numpy as jnp
from jax import lax
from jax.experimental import pallas as pl
from jax.experimental.pallas import tpu as pltpu
from jax.experimental.pallas import tpu_sc as plsc

F32 = jnp.float32
I32 = jnp.int32

_NUM_CORES = 2
_NUM_TILES = 16


def _mm1_body(q_ref, w_ref, o_ref):
    o_ref[...] = jnp.dot(q_ref[...], w_ref[...], preferred_element_type=F32)


def _mm1(qflat, W1):
    M, D = qflat.shape
    BLK = 2000
    return pl.pallas_call(
        _mm1_body,
        grid=(M // BLK,),
        in_specs=[pl.BlockSpec((BLK, D), lambda i: (i, 0)),
                  pl.BlockSpec((D, W1.shape[1]), lambda i: (0, 0))],
        out_specs=pl.BlockSpec((BLK, W1.shape[1]), lambda i: (i, 0)),
        out_shape=jax.ShapeDtypeStruct((M, W1.shape[1]), F32),
    )(qflat, W1)


def _sc1_build(NB, NP, DM, EPT, NG, G, STRIPE):
    mesh = plsc.VectorSubcoreMesh(core_axis_name="c", subcore_axis_name="s",
                                  num_cores=_NUM_CORES, num_subcores=_NUM_TILES)

    @functools.partial(
        pl.kernel,
        out_type=jax.ShapeDtypeStruct((NB, NP, DM), F32),
        mesh=mesh,
        scratch_types=[
            pltpu.VMEM((2, G), I32),
            pltpu.VMEM((2, G), I32),
            pltpu.VMEM((G, DM), F32),
            pltpu.VMEM((G, DM), F32),
            pltpu.VMEM_SHARED((NP, DM), F32),
            pltpu.SemaphoreType.DMA,
            pltpu.SemaphoreType.DMA,
            pltpu.SemaphoreType.DMA,
            pltpu.SemaphoreType.DMA,
        ],
    )
    def k(h_hbm, sd_hbm, z_hbm, out_hbm,
          sdA, sdB, rowsA, rowsB, acc, gA, gB, sA, sB):
        c = lax.axis_index("c")
        s = lax.axis_index("s")
        row0 = s * STRIPE
        for bb in range(NB // _NUM_CORES):
            b = c * (NB // _NUM_CORES) + bb
            pltpu.sync_copy(z_hbm, acc.at[pl.ds(row0, STRIPE)])
            plsc.subcore_barrier()

            def body(j, carry):
                gi = s * NG + j * 2

                @pl.when(j > 0)
                def _drainA():
                    pltpu.make_async_copy(z_hbm.at[pl.ds(0, G)], rowsA, sA).wait()

                pltpu.sync_copy(sd_hbm.at[b, gi], sdA)
                pltpu.async_copy(h_hbm.at[sdA.at[0]], rowsA, gA).wait()
                pltpu.async_copy(rowsA, acc.at[sdA.at[1]], sA, add=True)

                @pl.when(j > 0)
                def _drainB():
                    pltpu.make_async_copy(z_hbm.at[pl.ds(0, G)], rowsB, sB).wait()

                pltpu.sync_copy(sd_hbm.at[b, gi + 1], sdB)
                pltpu.async_copy(h_hbm.at[sdB.at[0]], rowsB, gB).wait()
                pltpu.async_copy(rowsB, acc.at[sdB.at[1]], sB, add=True)
                return carry

            lax.fori_loop(0, NG // 2, body, 0)
            pltpu.make_async_copy(z_hbm.at[pl.ds(0, G)], rowsA, sA).wait()
            pltpu.make_async_copy(z_hbm.at[pl.ds(0, G)], rowsB, sB).wait()
            plsc.subcore_barrier()
            pltpu.sync_copy(acc.at[pl.ds(row0, STRIPE)],
                            out_hbm.at[b, pl.ds(row0, STRIPE)])

    return k


def _mm2_body(a_ref, b1_ref, w_ref, o_ref):
    t = jnp.tanh(a_ref[...] + b1_ref[...])
    o_ref[...] = jnp.sum(t * w_ref[...], axis=1, keepdims=True)


def _mm2(aggflat, b1, W2):
    M, D = aggflat.shape
    BLK = next(b for b in (4096, 2048, 1024, 512, 256, 128, 64, 32, 16, 8)
               if M % b == 0)
    return pl.pallas_call(
        _mm2_body,
        grid=(M // BLK,),
        in_specs=[pl.BlockSpec((BLK, D), lambda i: (i, 0)),
                  pl.BlockSpec((1, D), lambda i: (0, 0)),
                  pl.BlockSpec((1, D), lambda i: (0, 0))],
        out_specs=pl.BlockSpec((BLK, 1), lambda i: (i, 0)),
        out_shape=jax.ShapeDtypeStruct((M, 1), F32),
    )(aggflat, b1.reshape(1, D), W2.reshape(1, D))


def _sc2_build(NB, NP, EPT):
    NPB = NP * NB
    NCH = EPT // 16
    NW = _NUM_CORES * _NUM_TILES
    mesh = plsc.VectorSubcoreMesh(core_axis_name="c", subcore_axis_name="s",
                                  num_cores=_NUM_CORES, num_subcores=_NUM_TILES)

    @functools.partial(
        pl.kernel,
        out_type=jax.ShapeDtypeStruct((NW * NPB,), F32),
        mesh=mesh,
        compiler_params=pltpu.CompilerParams(needs_layout_passes=False),
        scratch_types=[
            pltpu.VMEM((NPB,), F32),
            pltpu.VMEM((NPB,), F32),
            pltpu.VMEM((EPT,), I32),
            pltpu.VMEM((EPT,), I32),
        ],
    )
    def k(s_hbm, src_hbm, dst_hbm, z_hbm, out_hbm,
          s_loc, acc, src_loc, dst_loc):
        c = lax.axis_index("c")
        s = lax.axis_index("s")
        w = c * _NUM_TILES + s
        pltpu.sync_copy(s_hbm, s_loc)
        pltpu.sync_copy(z_hbm, acc)
        pltpu.sync_copy(src_hbm.at[pl.ds(w * EPT, EPT)], src_loc)
        pltpu.sync_copy(dst_hbm.at[pl.ds(w * EPT, EPT)], dst_loc)

        def body(i, carry):
            sv = src_loc[pl.ds(i * 16, 16)] * NB
            dv = dst_loc[pl.ds(i * 16, 16)] * NB
            for b in range(NB):
                vals = plsc.load_gather(s_loc, [sv + b])
                plsc.addupdate_scatter(acc, [dv + b], vals)
            return carry

        lax.fori_loop(0, NCH, body, 0)
        pltpu.sync_copy(acc, out_hbm.at[pl.ds(w * NPB, NPB)])

    return k


def _att_body(NB, NVALID, pt_ref, b2_ref, v_ref, ctx_ref, sc_ref):
    NW = pt_ref.shape[0]
    sp = pt_ref[0] + b2_ref[...]
    for t in range(1, NW):
        sp = sp + pt_ref[t]
    sc_ref[...] = sp
    col = lax.broadcasted_iota(I32, sp.shape, 1)
    valid = col < NVALID
    m = jnp.max(jnp.where(valid, sp, -1e30), axis=1, keepdims=True)
    e = jnp.where(valid, jnp.exp(sp - m), 0.0)
    z = jnp.sum(e, axis=1, keepdims=True)
    attn = e / z
    ctx_ref[...] = jnp.concatenate(
        [jnp.dot(attn[b:b + 1, :NVALID], v_ref[b],
                 preferred_element_type=F32,
                 precision=lax.Precision.HIGHEST)
         for b in range(NB)], axis=0)


def _att(Pt, b2, values):
    NB, N, D = values.shape
    NW, NP = Pt.shape[0], Pt.shape[2]
    return pl.pallas_call(
        functools.partial(_att_body, NB, N),
        grid=(1,),
        in_specs=[pl.BlockSpec((NW, NB, NP), lambda i: (0, 0, 0)),
                  pl.BlockSpec((1, 1), lambda i: (0, 0)),
                  pl.BlockSpec((NB, N, D), lambda i: (0, 0, 0))],
        out_specs=[pl.BlockSpec((NB, D), lambda i: (0, 0)),
                   pl.BlockSpec((NB, NP), lambda i: (0, 0))],
        out_shape=[jax.ShapeDtypeStruct((NB, D), F32),
                   jax.ShapeDtypeStruct((NB, NP), F32)],
    )(Pt, b2.reshape(1, 1), values)


def kernel(query, values, edges, W1, b1, W2, b2):
    NB, N, D = query.shape
    E = edges.shape[1]

    NP = -(-(N + 1) // 2048) * 2048
    STRIPE = NP // _NUM_TILES
    G1, G2 = 128, 64
    EPT1 = -(-E // (_NUM_TILES * G1)) * G1
    if (EPT1 // G1) % 2:
        EPT1 += G1
    E_pad = EPT1 * _NUM_TILES
    NG1 = EPT1 // G1
    EPT2 = E_pad // (_NUM_TILES * _NUM_CORES)
    NG2 = EPT2 // G2

    src = edges[0]
    dst = edges[1]
    pad = E_pad - E
    src_p = jnp.concatenate([src, jnp.arange(pad, dtype=I32) % N])
    dst_p = jnp.concatenate([dst, N + (jnp.arange(pad, dtype=I32) % (NP - N))])
    srcb = src_p[None, :] + (jnp.arange(NB, dtype=I32) * N)[:, None]
    NGT = E_pad // G1
    sd4 = jnp.stack([srcb.reshape(NB, NGT, G1),
                     jnp.broadcast_to(dst_p.reshape(NGT, G1),
                                      (NB, NGT, G1))], axis=2)
    z1 = jnp.zeros((STRIPE, D), F32)
    z4 = jnp.zeros((NP * NB,), F32)
    NW = _NUM_CORES * _NUM_TILES

    Hflat = _mm1(query.reshape(NB * N, D), W1)
    AGG = _sc1_build(NB, NP, D, EPT1, NG1, G1, STRIPE)(
        Hflat, sd4, z1)
    Sv = _mm2(AGG.reshape(NB * NP, D), b1, W2)
    S4 = Sv.reshape(NB, NP).T.reshape(NP * NB)
    P = _sc2_build(NB, NP, EPT2)(S4, src_p, dst_p, z4)
    Pt = jnp.transpose(P.reshape(NW, NP, NB), (0, 2, 1))
    ctx, scores = _att(Pt, b2, values)
    return ctx, scores[:, :N, None]


# speedup vs baseline: 3.3449x; 1.0644x over previous
import functools

import jax
import jax.numpy as jnp
from jax import lax
from jax.experimental import pallas as pl
from jax.experimental.pallas import tpu as pltpu
from jax.experimental.pallas import tpu_sc as plsc

F32 = jnp.float32
I32 = jnp.int32

_NUM_CORES = 2
_NUM_TILES = 16


def _mm1_body(q_ref, w_ref, o_ref):
    o_ref[...] = jnp.dot(q_ref[...], w_ref[...], preferred_element_type=F32)


def _mm1(qflat, W1):
    M, D = qflat.shape
    BLK = 2000
    return pl.pallas_call(
        _mm1_body,
        grid=(M // BLK,),
        in_specs=[pl.BlockSpec((BLK, D), lambda i: (i, 0)),
                  pl.BlockSpec((D, W1.shape[1]), lambda i: (0, 0))],
        out_specs=pl.BlockSpec((BLK, W1.shape[1]), lambda i: (i, 0)),
        out_shape=jax.ShapeDtypeStruct((M, W1.shape[1]), F32),
    )(qflat, W1)


def _sc1_build(NB, NP, DM, EPT, NG, G, STRIPE):
    mesh = plsc.VectorSubcoreMesh(core_axis_name="c", subcore_axis_name="s",
                                  num_cores=_NUM_CORES, num_subcores=_NUM_TILES)

    @functools.partial(
        pl.kernel,
        out_type=jax.ShapeDtypeStruct((NB, NP, DM), F32),
        mesh=mesh,
        scratch_types=[
            pltpu.VMEM((2, 2, G), I32),
            pltpu.VMEM((2, 2, G), I32),
            pltpu.VMEM((G, DM), F32),
            pltpu.VMEM((G, DM), F32),
            pltpu.VMEM_SHARED((NP, DM), F32),
            pltpu.SemaphoreType.DMA,
            pltpu.SemaphoreType.DMA,
            pltpu.SemaphoreType.DMA,
            pltpu.SemaphoreType.DMA,
        ],
    )
    def k(h_hbm, sd_hbm, z_hbm, out_hbm,
          sdA, sdB, rowsA, rowsB, acc, gA, gB, sA, sB):
        c = lax.axis_index("c")
        s = lax.axis_index("s")
        row0 = s * STRIPE
        for bb in range(NB // _NUM_CORES):
            b = c * (NB // _NUM_CORES) + bb
            pltpu.sync_copy(z_hbm, acc.at[pl.ds(row0, STRIPE)])
            plsc.subcore_barrier()

            def body(j, carry):
                jj = j % 2

                @pl.when(j > 0)
                def _drainA():
                    pltpu.make_async_copy(z_hbm.at[pl.ds(0, G)], rowsA, sA).wait()

                @pl.when(jj == 0)
                def _loadA():
                    pltpu.sync_copy(sd_hbm.at[b, s, 0, pl.ds(j, 2)], sdA)

                pltpu.async_copy(h_hbm.at[sdA.at[jj, 0]], rowsA, gA).wait()
                pltpu.async_copy(rowsA, acc.at[sdA.at[jj, 1]], sA, add=True)

                @pl.when(j > 0)
                def _drainB():
                    pltpu.make_async_copy(z_hbm.at[pl.ds(0, G)], rowsB, sB).wait()

                @pl.when(jj == 0)
                def _loadB():
                    pltpu.sync_copy(sd_hbm.at[b, s, 1, pl.ds(j, 2)], sdB)

                pltpu.async_copy(h_hbm.at[sdB.at[jj, 0]], rowsB, gB).wait()
                pltpu.async_copy(rowsB, acc.at[sdB.at[jj, 1]], sB, add=True)
                return carry

            lax.fori_loop(0, NG // 2, body, 0)
            pltpu.make_async_copy(z_hbm.at[pl.ds(0, G)], rowsA, sA).wait()
            pltpu.make_async_copy(z_hbm.at[pl.ds(0, G)], rowsB, sB).wait()
            plsc.subcore_barrier()
            pltpu.sync_copy(acc.at[pl.ds(row0, STRIPE)],
                            out_hbm.at[b, pl.ds(row0, STRIPE)])

    return k


def _mm2_body(a_ref, b1_ref, w_ref, o_ref):
    t = jnp.tanh(a_ref[...] + b1_ref[...])
    o_ref[...] = jnp.sum(t * w_ref[...], axis=1, keepdims=True)


def _mm2(aggflat, b1, W2):
    M, D = aggflat.shape
    BLK = next(b for b in (4096, 2048, 1024, 512, 256, 128, 64, 32, 16, 8)
               if M % b == 0)
    return pl.pallas_call(
        _mm2_body,
        grid=(M // BLK,),
        in_specs=[pl.BlockSpec((BLK, D), lambda i: (i, 0)),
                  pl.BlockSpec((1, D), lambda i: (0, 0)),
                  pl.BlockSpec((1, D), lambda i: (0, 0))],
        out_specs=pl.BlockSpec((BLK, 1), lambda i: (i, 0)),
        out_shape=jax.ShapeDtypeStruct((M, 1), F32),
    )(aggflat, b1.reshape(1, D), W2.reshape(1, D))


def _sc2_build(NB, NP, EPT):
    NPB = NP * NB
    NCH = EPT // 16
    NW = _NUM_CORES * _NUM_TILES
    mesh = plsc.VectorSubcoreMesh(core_axis_name="c", subcore_axis_name="s",
                                  num_cores=_NUM_CORES, num_subcores=_NUM_TILES)

    @functools.partial(
        pl.kernel,
        out_type=jax.ShapeDtypeStruct((NW * NPB,), F32),
        mesh=mesh,
        compiler_params=pltpu.CompilerParams(needs_layout_passes=False),
        scratch_types=[
            pltpu.VMEM((NPB,), F32),
            pltpu.VMEM((NPB,), F32),
            pltpu.VMEM((EPT,), I32),
            pltpu.VMEM((EPT,), I32),
        ],
    )
    def k(s_hbm, src_hbm, dst_hbm, z_hbm, out_hbm,
          s_loc, acc, src_loc, dst_loc):
        c = lax.axis_index("c")
        s = lax.axis_index("s")
        w = c * _NUM_TILES + s
        pltpu.sync_copy(s_hbm, s_loc)
        pltpu.sync_copy(z_hbm, acc)
        pltpu.sync_copy(src_hbm.at[pl.ds(w * EPT, EPT)], src_loc)
        pltpu.sync_copy(dst_hbm.at[pl.ds(w * EPT, EPT)], dst_loc)

        def body(i, carry):
            sv = src_loc[pl.ds(i * 16, 16)] * NB
            dv = dst_loc[pl.ds(i * 16, 16)] * NB
            for b in range(NB):
                vals = plsc.load_gather(s_loc, [sv + b])
                plsc.addupdate_scatter(acc, [dv + b], vals)
            return carry

        lax.fori_loop(0, NCH, body, 0)
        pltpu.sync_copy(acc, out_hbm.at[pl.ds(w * NPB, NPB)])

    return k


def _att_body(NB, NVALID, pt_ref, b2_ref, v_ref, ctx_ref, sc_ref):
    NW = pt_ref.shape[0]
    sp = pt_ref[0] + b2_ref[...]
    for t in range(1, NW):
        sp = sp + pt_ref[t]
    sc_ref[...] = sp
    col = lax.broadcasted_iota(I32, sp.shape, 1)
    valid = col < NVALID
    m = jnp.max(jnp.where(valid, sp, -1e30), axis=1, keepdims=True)
    e = jnp.where(valid, jnp.exp(sp - m), 0.0)
    z = jnp.sum(e, axis=1, keepdims=True)
    attn = e / z
    ctx_ref[...] = jnp.concatenate(
        [jnp.dot(attn[b:b + 1, :NVALID], v_ref[b],
                 preferred_element_type=F32,
                 precision=lax.Precision.HIGHEST)
         for b in range(NB)], axis=0)


def _att(Pt, b2, values):
    NB, N, D = values.shape
    NW, NP = Pt.shape[0], Pt.shape[2]
    return pl.pallas_call(
        functools.partial(_att_body, NB, N),
        grid=(1,),
        in_specs=[pl.BlockSpec((NW, NB, NP), lambda i: (0, 0, 0)),
                  pl.BlockSpec((1, 1), lambda i: (0, 0)),
                  pl.BlockSpec((NB, N, D), lambda i: (0, 0, 0))],
        out_specs=[pl.BlockSpec((NB, D), lambda i: (0, 0)),
                   pl.BlockSpec((NB, NP), lambda i: (0, 0))],
        out_shape=[jax.ShapeDtypeStruct((NB, D), F32),
                   jax.ShapeDtypeStruct((NB, NP), F32)],
    )(Pt, b2.reshape(1, 1), values)


def kernel(query, values, edges, W1, b1, W2, b2):
    NB, N, D = query.shape
    E = edges.shape[1]

    NP = -(-(N + 1) // 2048) * 2048
    STRIPE = NP // _NUM_TILES
    G1, G2 = 128, 64
    EPT1 = -(-E // (_NUM_TILES * G1)) * G1
    while (EPT1 // G1) % 4:
        EPT1 += G1
    E_pad = EPT1 * _NUM_TILES
    NG1 = EPT1 // G1
    EPT2 = E_pad // (_NUM_TILES * _NUM_CORES)
    NG2 = EPT2 // G2

    src = edges[0]
    dst = edges[1]
    pad = E_pad - E
    src_p = jnp.concatenate([src, jnp.arange(pad, dtype=I32) % N])
    dst_p = jnp.concatenate([dst, N + (jnp.arange(pad, dtype=I32) % (NP - N))])
    srcb = src_p[None, :] + (jnp.arange(NB, dtype=I32) * N)[:, None]
    NT, NG1h = _NUM_TILES, NG1 // 2
    sd = jnp.stack([srcb.reshape(NB, NT, NG1, G1),
                    jnp.broadcast_to(dst_p.reshape(NT, NG1, G1),
                                     (NB, NT, NG1, G1))], axis=3)
    sd5 = sd.reshape(NB, NT, NG1h, 2, 2, G1).transpose(0, 1, 3, 2, 4, 5)
    z1 = jnp.zeros((STRIPE, D), F32)
    z4 = jnp.zeros((NP * NB,), F32)
    NW = _NUM_CORES * _NUM_TILES

    Hflat = _mm1(query.reshape(NB * N, D), W1)
    AGG = _sc1_build(NB, NP, D, EPT1, NG1, G1, STRIPE)(
        Hflat, sd5, z1)
    Sv = _mm2(AGG.reshape(NB * NP, D), b1, W2)
    S4 = Sv.reshape(NB, NP).T.reshape(NP * NB)
    P = _sc2_build(NB, NP, EPT2)(S4, src_p, dst_p, z4)
    Pt = jnp.transpose(P.reshape(NW, NP, NB), (0, 2, 1))
    ctx, scores = _att(Pt, b2, values)
    return ctx, scores[:, :N, None]


# speedup vs baseline: 3.6374x; 1.0875x over previous
import functools

import jax
import jax.numpy as jnp
from jax import lax
from jax.experimental import pallas as pl
from jax.experimental.pallas import tpu as pltpu
from jax.experimental.pallas import tpu_sc as plsc

F32 = jnp.float32
I32 = jnp.int32

_NUM_CORES = 2
_NUM_TILES = 16


def _mm1_body(q_ref, w_ref, o_ref):
    o_ref[...] = jnp.dot(q_ref[...], w_ref[...], preferred_element_type=F32)


def _mm1(qflat, W1):
    M, D = qflat.shape
    BLK = 2000
    return pl.pallas_call(
        _mm1_body,
        grid=(M // BLK,),
        in_specs=[pl.BlockSpec((BLK, D), lambda i: (i, 0)),
                  pl.BlockSpec((D, W1.shape[1]), lambda i: (0, 0))],
        out_specs=pl.BlockSpec((BLK, W1.shape[1]), lambda i: (i, 0)),
        out_shape=jax.ShapeDtypeStruct((M, W1.shape[1]), F32),
    )(qflat, W1)


def _sc1_build(NB, NP, DM, EPT, NG, G, STRIPE):
    mesh = plsc.VectorSubcoreMesh(core_axis_name="c", subcore_axis_name="s",
                                  num_cores=_NUM_CORES, num_subcores=_NUM_TILES)

    @functools.partial(
        pl.kernel,
        out_type=jax.ShapeDtypeStruct((NB, NP, DM), F32),
        mesh=mesh,
        scratch_types=[
            pltpu.VMEM((4, 2, G), I32),
            pltpu.VMEM((4, 2, G), I32),
            pltpu.VMEM((G, DM), F32),
            pltpu.VMEM((G, DM), F32),
            pltpu.VMEM_SHARED((NP, DM), F32),
            pltpu.SemaphoreType.DMA,
            pltpu.SemaphoreType.DMA,
            pltpu.SemaphoreType.DMA,
            pltpu.SemaphoreType.DMA,
        ],
    )
    def k(h_hbm, sd_hbm, z_hbm, out_hbm,
          sdA, sdB, rowsA, rowsB, acc, gA, gB, sA, sB):
        c = lax.axis_index("c")
        s = lax.axis_index("s")
        row0 = s * STRIPE
        for bb in range(NB // _NUM_CORES):
            b = c * (NB // _NUM_CORES) + bb
            pltpu.sync_copy(z_hbm, acc.at[pl.ds(row0, STRIPE)])
            plsc.subcore_barrier()

            def body(j, carry):
                jj = j % 4

                @pl.when(j > 0)
                def _drainA():
                    pltpu.make_async_copy(z_hbm.at[pl.ds(0, G)], rowsA, sA).wait()

                @pl.when(jj == 0)
                def _loadA():
                    pltpu.sync_copy(sd_hbm.at[b, s, 0, pl.ds(j, 4)], sdA)

                dA = pltpu.async_copy(h_hbm.at[sdA.at[jj, 0]], rowsA, gA)

                @pl.when(j > 0)
                def _drainB():
                    pltpu.make_async_copy(z_hbm.at[pl.ds(0, G)], rowsB, sB).wait()

                @pl.when(jj == 0)
                def _loadB():
                    pltpu.sync_copy(sd_hbm.at[b, s, 1, pl.ds(j, 4)], sdB)

                dB = pltpu.async_copy(h_hbm.at[sdB.at[jj, 0]], rowsB, gB)
                dA.wait()
                pltpu.async_copy(rowsA, acc.at[sdA.at[jj, 1]], sA, add=True)
                dB.wait()
                pltpu.async_copy(rowsB, acc.at[sdB.at[jj, 1]], sB, add=True)
                return carry

            lax.fori_loop(0, NG // 2, body, 0)
            pltpu.make_async_copy(z_hbm.at[pl.ds(0, G)], rowsA, sA).wait()
            pltpu.make_async_copy(z_hbm.at[pl.ds(0, G)], rowsB, sB).wait()
            plsc.subcore_barrier()
            pltpu.sync_copy(acc.at[pl.ds(row0, STRIPE)],
                            out_hbm.at[b, pl.ds(row0, STRIPE)])

    return k


def _mm2_body(a_ref, b1_ref, w_ref, o_ref):
    t = jnp.tanh(a_ref[...] + b1_ref[...])
    o_ref[...] = jnp.sum(t * w_ref[...], axis=1, keepdims=True)


def _mm2(aggflat, b1, W2):
    M, D = aggflat.shape
    BLK = next(b for b in (4096, 2048, 1024, 512, 256, 128, 64, 32, 16, 8)
               if M % b == 0)
    return pl.pallas_call(
        _mm2_body,
        grid=(M // BLK,),
        in_specs=[pl.BlockSpec((BLK, D), lambda i: (i, 0)),
                  pl.BlockSpec((1, D), lambda i: (0, 0)),
                  pl.BlockSpec((1, D), lambda i: (0, 0))],
        out_specs=pl.BlockSpec((BLK, 1), lambda i: (i, 0)),
        out_shape=jax.ShapeDtypeStruct((M, 1), F32),
    )(aggflat, b1.reshape(1, D), W2.reshape(1, D))


def _sc2_build(NB, NP, EPT):
    NPB = NP * NB
    NCH = EPT // 16
    NW = _NUM_CORES * _NUM_TILES
    mesh = plsc.VectorSubcoreMesh(core_axis_name="c", subcore_axis_name="s",
                                  num_cores=_NUM_CORES, num_subcores=_NUM_TILES)

    @functools.partial(
        pl.kernel,
        out_type=jax.ShapeDtypeStruct((NW * NPB,), F32),
        mesh=mesh,
        compiler_params=pltpu.CompilerParams(needs_layout_passes=False),
        scratch_types=[
            pltpu.VMEM((NPB,), F32),
            pltpu.VMEM((NPB,), F32),
            pltpu.VMEM((EPT,), I32),
            pltpu.VMEM((EPT,), I32),
        ],
    )
    def k(s_hbm, src_hbm, dst_hbm, z_hbm, out_hbm,
          s_loc, acc, src_loc, dst_loc):
        c = lax.axis_index("c")
        s = lax.axis_index("s")
        w = c * _NUM_TILES + s
        pltpu.sync_copy(s_hbm, s_loc)
        pltpu.sync_copy(z_hbm, acc)
        pltpu.sync_copy(src_hbm.at[pl.ds(w * EPT, EPT)], src_loc)
        pltpu.sync_copy(dst_hbm.at[pl.ds(w * EPT, EPT)], dst_loc)

        def body(i, carry):
            sv = src_loc[pl.ds(i * 16, 16)] * NB
            dv = dst_loc[pl.ds(i * 16, 16)] * NB
            for b in range(NB):
                vals = plsc.load_gather(s_loc, [sv + b])
                plsc.addupdate_scatter(acc, [dv + b], vals)
            return carry

        lax.fori_loop(0, NCH, body, 0)
        pltpu.sync_copy(acc, out_hbm.at[pl.ds(w * NPB, NPB)])

    return k


def _att_body(NB, NVALID, pt_ref, b2_ref, v_ref, ctx_ref, sc_ref):
    NW = pt_ref.shape[0]
    sp = pt_ref[0] + b2_ref[...]
    for t in range(1, NW):
        sp = sp + pt_ref[t]
    sc_ref[...] = sp
    col = lax.broadcasted_iota(I32, sp.shape, 1)
    valid = col < NVALID
    m = jnp.max(jnp.where(valid, sp, -1e30), axis=1, keepdims=True)
    e = jnp.where(valid, jnp.exp(sp - m), 0.0)
    z = jnp.sum(e, axis=1, keepdims=True)
    attn = e / z
    ctx_ref[...] = jnp.concatenate(
        [jnp.dot(attn[b:b + 1, :NVALID], v_ref[b],
                 preferred_element_type=F32,
                 precision=lax.Precision.HIGHEST)
         for b in range(NB)], axis=0)


def _att(Pt, b2, values):
    NB, N, D = values.shape
    NW, NP = Pt.shape[0], Pt.shape[2]
    return pl.pallas_call(
        functools.partial(_att_body, NB, N),
        grid=(1,),
        in_specs=[pl.BlockSpec((NW, NB, NP), lambda i: (0, 0, 0)),
                  pl.BlockSpec((1, 1), lambda i: (0, 0)),
                  pl.BlockSpec((NB, N, D), lambda i: (0, 0, 0))],
        out_specs=[pl.BlockSpec((NB, D), lambda i: (0, 0)),
                   pl.BlockSpec((NB, NP), lambda i: (0, 0))],
        out_shape=[jax.ShapeDtypeStruct((NB, D), F32),
                   jax.ShapeDtypeStruct((NB, NP), F32)],
    )(Pt, b2.reshape(1, 1), values)


def kernel(query, values, edges, W1, b1, W2, b2):
    NB, N, D = query.shape
    E = edges.shape[1]

    NP = -(-(N + 1) // 2048) * 2048
    STRIPE = NP // _NUM_TILES
    G1, G2 = 128, 64
    EPT1 = -(-E // (_NUM_TILES * G1)) * G1
    while (EPT1 // G1) % 4:
        EPT1 += G1
    E_pad = EPT1 * _NUM_TILES
    NG1 = EPT1 // G1
    EPT2 = E_pad // (_NUM_TILES * _NUM_CORES)
    NG2 = EPT2 // G2

    src = edges[0]
    dst = edges[1]
    pad = E_pad - E
    src_p = jnp.concatenate([src, jnp.arange(pad, dtype=I32) % N])
    dst_p = jnp.concatenate([dst, N + (jnp.arange(pad, dtype=I32) % (NP - N))])
    srcb = src_p[None, :] + (jnp.arange(NB, dtype=I32) * N)[:, None]
    NT, NG1h = _NUM_TILES, NG1 // 2
    sd = jnp.stack([srcb.reshape(NB, NT, NG1, G1),
                    jnp.broadcast_to(dst_p.reshape(NT, NG1, G1),
                                     (NB, NT, NG1, G1))], axis=3)
    sd5 = sd.reshape(NB, NT, NG1h, 2, 2, G1).transpose(0, 1, 3, 2, 4, 5)
    z1 = jnp.zeros((STRIPE, D), F32)
    z4 = jnp.zeros((NP * NB,), F32)
    NW = _NUM_CORES * _NUM_TILES

    Hflat = _mm1(query.reshape(NB * N, D), W1)
    AGG = _sc1_build(NB, NP, D, EPT1, NG1, G1, STRIPE)(
        Hflat, sd5, z1)
    Sv = _mm2(AGG.reshape(NB * NP, D), b1, W2)
    S4 = Sv.reshape(NB, NP).T.reshape(NP * NB)
    P = _sc2_build(NB, NP, EPT2)(S4, src_p, dst_p, z4)
    Pt = jnp.transpose(P.reshape(NW, NP, NB), (0, 2, 1))
    ctx, scores = _att(Pt, b2, values)
    return ctx, scores[:, :N, None]
